# baseline (speedup 1.0000x reference)
import functools

import jax
import jax.numpy as jnp
from jax import lax
from jax.experimental import pallas as pl
from jax.experimental.pallas import tpu as pltpu
from jax.experimental.pallas import tpu_sc as plsc

N = 20000
R = 160
PAD = R * 128
K_PRE = 6000
K_POST = 1000
THR = 0.7

RC = 48
C_PAD = RC * 128
SPARE = C_PAD + 64
OUT_ROWS = SPARE + PAD

NW = 32
CHUNK = PAD // NW
CB = CHUNK // 128
BIGI = 2 ** 30


def _select_body(s_ref, dest_ref):
    s = s_ref[...]
    u = lax.bitcast_convert_type(s, jnp.uint32)
    key = jnp.where(s < 0, ~u, u | jnp.uint32(0x80000000))

    prefix = jnp.uint32(0)
    for b in range(31, -1, -1):
        cand = prefix | jnp.uint32(1 << b)
        cnt = jnp.sum((key >= cand).astype(jnp.int32))
        prefix = jnp.where(cnt >= K_PRE, cand, prefix)

    cnt_gt = jnp.sum((key > prefix).astype(jnp.int32))
    tie = key == prefix
    tie_f = tie.astype(jnp.float32)
    incl = (lax.broadcasted_iota(jnp.int32, (128, 128), 0)
            <= lax.broadcasted_iota(jnp.int32, (128, 128), 1)).astype(jnp.float32)
    strict = (lax.broadcasted_iota(jnp.int32, (R, R), 1)
              < lax.broadcasted_iota(jnp.int32, (R, R), 0)).astype(jnp.float32)

    tcum = jnp.dot(tie_f, incl, preferred_element_type=jnp.float32)
    row_off = jnp.dot(strict, tcum[:, 127:128], preferred_element_type=jnp.float32)
    ordinal = row_off + tcum - tie_f
    need = (K_PRE - cnt_gt).astype(jnp.float32)
    is_cand = (key > prefix) | (tie & (ordinal < need))

    cand_f = is_cand.astype(jnp.float32)
    ccum = jnp.dot(cand_f, incl, preferred_element_type=jnp.float32)
    crow_off = jnp.dot(strict, ccum[:, 127:128], preferred_element_type=jnp.float32)
    pos = crow_off + ccum - cand_f
    lin = (lax.broadcasted_iota(jnp.int32, (R, 128), 0) * 128
           + lax.broadcasted_iota(jnp.int32, (R, 128), 1))
    dest_ref[...] = jnp.where(is_cand, pos.astype(jnp.int32), SPARE + lin)


def _compact_body(rows_h, dest_h, out_h, rows_v, dest_v, sem):
    wid = lax.axis_index("s") * 2 + lax.axis_index("c")
    pltpu.sync_copy(rows_h.at[wid], rows_v)
    pltpu.sync_copy(dest_h.at[wid], dest_v)
    copies = [pltpu.make_async_copy(rows_v.at[j], out_h.at[dest_v.at[j]], sem)
              for j in range(CB)]
    for c in copies:
        c.start()
    for c in copies:
        c.wait()


W = 12


def _nms_body(x1_ref, y1_ref, x2_ref, y2_ref, s_ref, packed_ref,
              ox1, oy1, ox2, oy2, osc,
              ms_ref, area_ref, lin_ref):
    lin = (lax.broadcasted_iota(jnp.int32, (RC, 128), 0) * 128
           + lax.broadcasted_iota(jnp.int32, (RC, 128), 1))
    lin_ref[...] = lin
    ms_ref[...] = jnp.where(lin < K_PRE, s_ref[...], -jnp.inf)
    x1 = x1_ref[...]
    y1 = y1_ref[...]
    x2 = x2_ref[...]
    y2 = y2_ref[...]
    area_ref[...] = jnp.maximum(x2 - x1, 0.0) * jnp.maximum(y2 - y1, 0.0)
    zeros = jnp.zeros((K_POST + W, 1), jnp.float32)
    ox1[...] = zeros
    oy1[...] = zeros
    ox2[...] = zeros
    oy2[...] = zeros
    osc[...] = zeros

    def rmax(a):
        return jnp.max(jnp.max(a, axis=1, keepdims=True), axis=0, keepdims=True)

    def rmin(a):
        return jnp.min(jnp.min(a, axis=1, keepdims=True), axis=0, keepdims=True)

    def rsum(a):
        return jnp.sum(jnp.sum(a, axis=1, keepdims=True), axis=0, keepdims=True)

    lane = lax.broadcasted_iota(jnp.int32, (1, 128), 1)

    def picks(j):
        js = j[0, 0]
        r = js >> 7
        lm = lane == (js & 127)
        p = jnp.sum(jnp.where(lm, packed_ref[pl.ds(5 * r, 5), :], 0.0),
                    axis=1, keepdims=True)
        return (p[0:1], p[1:2], p[2:3], p[3:4], p[4:5])

    def piou(a, b):
        xx1 = jnp.maximum(a[0], b[0])
        yy1 = jnp.maximum(a[1], b[1])
        xx2 = jnp.minimum(a[2], b[2])
        yy2 = jnp.minimum(a[3], b[3])
        inter = jnp.maximum(xx2 - xx1, 0.0) * jnp.maximum(yy2 - yy1, 0.0)
        return inter / (a[4] + b[4] - inter + 1e-9)

    one = jnp.int32(1)

    def step(carry):
        cnt, _ = carry
        ms = ms_ref[...]
        lin = lin_ref[...]
        m, e, c, j, b = [], [], [], [], []
        ms_cur = ms
        for w in range(W):
            mw = rmax(ms_cur)
            ew = ms_cur == mw
            m.append(mw)
            e.append(ew)
            if w < W - 1:
                ms_cur = jnp.where(ew, -jnp.inf, ms_cur)
                c.append(rsum(ew.astype(jnp.int32)))
            j.append(rmin(jnp.where(ew, lin, BIGI)))
            b.append(picks(j[w]))

        a = [m[0] > -jnp.inf]
        for w in range(1, W):
            acc = a[w - 1] & (c[w - 1] == one) & (m[w] > -jnp.inf)
            for v in range(w):
                acc = acc & jnp.logical_not(piou(b[v], b[w]) > THR)
            a.append(acc)

        def srow(a_w, j_w, b_w):
            xx1 = jnp.maximum(b_w[0], x1)
            yy1 = jnp.maximum(b_w[1], y1)
            xx2 = jnp.minimum(b_w[2], x2)
            yy2 = jnp.minimum(b_w[3], y2)
            inter = jnp.maximum(xx2 - xx1, 0.0) * jnp.maximum(yy2 - yy1, 0.0)
            iou = inter / (b_w[4] + area_ref[...] - inter + 1e-9)
            return a_w & ((iou > THR) | (lin == j_w))

        sup = srow(a[0], j[0], b[0])
        for w in range(1, W):
            sup = sup | srow(a[w], j[w], b[w])
        ms_ref[...] = jnp.where(sup, -jnp.inf, ms)

        for w in range(W):
            p = pl.ds(cnt + w, 1)
            ox1[p, :] = jnp.where(a[w], b[w][0], 0.0)
            oy1[p, :] = jnp.where(a[w], b[w][1], 0.0)
            ox2[p, :] = jnp.where(a[w], b[w][2], 0.0)
            oy2[p, :] = jnp.where(a[w], b[w][3], 0.0)
            osc[p, :] = jnp.where(a[w], m[w], 0.0)

        ka = a[0].astype(jnp.int32)
        for w in range(1, W):
            ka = ka + a[w].astype(jnp.int32)
        ka = ka[0, 0]
        return cnt + ka, ka == 0

    def cond(carry):
        cnt, done = carry
        return jnp.logical_and(cnt < K_POST, jnp.logical_not(done))

    lax.while_loop(cond, step, (jnp.int32(0), False))


def kernel(boxes, scores):
    s2 = jnp.pad(scores, (0, PAD - N), constant_values=-jnp.inf).reshape(R, 128)
    bx = jnp.pad(boxes, ((0, PAD - N), (0, 0)))
    aos = jnp.concatenate(
        [bx, s2.reshape(PAD, 1), jnp.zeros((PAD, 123), jnp.float32)], axis=1)

    dest = pl.pallas_call(
        _select_body,
        out_shape=jax.ShapeDtypeStruct((R, 128), jnp.int32),
    )(s2)

    mesh = plsc.VectorSubcoreMesh(core_axis_name="c", subcore_axis_name="s")
    compact = functools.partial(
        pl.kernel,
        out_type=jax.ShapeDtypeStruct((OUT_ROWS, 128), jnp.float32),
        scratch_types=[
            pltpu.VMEM((CB, 128, 128), jnp.float32),
            pltpu.VMEM((CB, 128), jnp.int32),
            pltpu.SemaphoreType.DMA,
        ],
        mesh=mesh,
    )(_compact_body)
    crows = compact(
        aos.reshape(NW, CB, 128, 128),
        dest.reshape(NW, CB, 128),
    )[:C_PAD]

    planes = [crows[:, i].reshape(RC, 128) for i in range(5)]
    careas = jnp.maximum(planes[2] - planes[0], 0.0) * jnp.maximum(
        planes[3] - planes[1], 0.0)
    packed = jnp.stack(planes[:4] + [careas], axis=1).reshape(5 * RC, 128)

    outs = pl.pallas_call(
        _nms_body,
        out_shape=[jax.ShapeDtypeStruct((K_POST + W, 1), jnp.float32)] * 5,
        scratch_shapes=[
            pltpu.VMEM((RC, 128), jnp.float32),
            pltpu.VMEM((RC, 128), jnp.float32),
            pltpu.VMEM((RC, 128), jnp.int32),
        ],
    )(*planes, packed)
    return jnp.concatenate([o[:K_POST] for o in outs], axis=1)


# speedup vs baseline: 1.0446x; 1.0446x over previous
import functools

import jax
import jax.numpy as jnp
from jax import lax
from jax.experimental import pallas as pl
from jax.experimental.pallas import tpu as pltpu
from jax.experimental.pallas import tpu_sc as plsc

N = 20000
R = 160
PAD = R * 128
K_PRE = 6000
K_POST = 1000
THR = 0.7

RC = 48
C_PAD = RC * 128
SPARE = C_PAD + 64
OUT_ROWS = SPARE + PAD

NW = 32
CHUNK = PAD // NW
CB = CHUNK // 128
BIGI = 2 ** 30


def _select_body(s_ref, dest_ref):
    s = s_ref[...]
    u = lax.bitcast_convert_type(s, jnp.uint32)
    key = jnp.where(s < 0, ~u, u | jnp.uint32(0x80000000))

    prefix = jnp.uint32(0)
    for b in range(31, -1, -1):
        cand = prefix | jnp.uint32(1 << b)
        cnt = jnp.sum((key >= cand).astype(jnp.int32))
        prefix = jnp.where(cnt >= K_PRE, cand, prefix)

    cnt_gt = jnp.sum((key > prefix).astype(jnp.int32))
    tie = key == prefix
    tie_f = tie.astype(jnp.float32)
    incl = (lax.broadcasted_iota(jnp.int32, (128, 128), 0)
            <= lax.broadcasted_iota(jnp.int32, (128, 128), 1)).astype(jnp.float32)
    strict = (lax.broadcasted_iota(jnp.int32, (R, R), 1)
              < lax.broadcasted_iota(jnp.int32, (R, R), 0)).astype(jnp.float32)

    tcum = jnp.dot(tie_f, incl, preferred_element_type=jnp.float32)
    row_off = jnp.dot(strict, tcum[:, 127:128], preferred_element_type=jnp.float32)
    ordinal = row_off + tcum - tie_f
    need = (K_PRE - cnt_gt).astype(jnp.float32)
    is_cand = (key > prefix) | (tie & (ordinal < need))

    cand_f = is_cand.astype(jnp.float32)
    ccum = jnp.dot(cand_f, incl, preferred_element_type=jnp.float32)
    crow_off = jnp.dot(strict, ccum[:, 127:128], preferred_element_type=jnp.float32)
    pos = crow_off + ccum - cand_f
    lin = (lax.broadcasted_iota(jnp.int32, (R, 128), 0) * 128
           + lax.broadcasted_iota(jnp.int32, (R, 128), 1))
    dest_ref[...] = jnp.where(is_cand, pos.astype(jnp.int32), SPARE + lin)


def _compact_body(rows_h, dest_h, out_h, rows_v, dest_v, sem):
    wid = lax.axis_index("s") * 2 + lax.axis_index("c")
    pltpu.sync_copy(rows_h.at[wid], rows_v)
    pltpu.sync_copy(dest_h.at[wid], dest_v)
    copies = [pltpu.make_async_copy(rows_v.at[j], out_h.at[dest_v.at[j]], sem)
              for j in range(CB)]
    for c in copies:
        c.start()
    for c in copies:
        c.wait()


W = 8


def _nms_body(x1_ref, y1_ref, x2_ref, y2_ref, s_ref, packed_ref,
              ox1, oy1, ox2, oy2, osc,
              ms_ref, area_ref, lin_ref):
    lin = (lax.broadcasted_iota(jnp.int32, (RC, 128), 0) * 128
           + lax.broadcasted_iota(jnp.int32, (RC, 128), 1))
    lin_ref[...] = lin
    ms_ref[...] = jnp.where(lin < K_PRE, s_ref[...], -jnp.inf)
    x1 = x1_ref[...]
    y1 = y1_ref[...]
    x2 = x2_ref[...]
    y2 = y2_ref[...]
    area_ref[...] = jnp.maximum(x2 - x1, 0.0) * jnp.maximum(y2 - y1, 0.0)
    zeros = jnp.zeros((K_POST + W, 1), jnp.float32)
    ox1[...] = zeros
    oy1[...] = zeros
    ox2[...] = zeros
    oy2[...] = zeros
    osc[...] = zeros

    def rmax(a):
        return jnp.max(jnp.max(a, axis=1, keepdims=True), axis=0, keepdims=True)

    def rmin(a):
        return jnp.min(jnp.min(a, axis=1, keepdims=True), axis=0, keepdims=True)

    def rsum(a):
        return jnp.sum(jnp.sum(a, axis=1, keepdims=True), axis=0, keepdims=True)

    lane = lax.broadcasted_iota(jnp.int32, (1, 128), 1)

    def picks(j):
        js = j[0, 0]
        r = js >> 7
        lm = lane == (js & 127)
        p = jnp.sum(jnp.where(lm, packed_ref[pl.ds(5 * r, 5), :], 0.0),
                    axis=1, keepdims=True)
        return (p[0:1], p[1:2], p[2:3], p[3:4], p[4:5])

    def piou(a, b):
        xx1 = jnp.maximum(a[0], b[0])
        yy1 = jnp.maximum(a[1], b[1])
        xx2 = jnp.minimum(a[2], b[2])
        yy2 = jnp.minimum(a[3], b[3])
        inter = jnp.maximum(xx2 - xx1, 0.0) * jnp.maximum(yy2 - yy1, 0.0)
        return inter / (a[4] + b[4] - inter + 1e-9)

    one = jnp.int32(1)

    def step(carry):
        cnt, _ = carry
        ms = ms_ref[...]
        lin = lin_ref[...]
        m, e, c, j, b = [], [], [], [], []
        ms_cur = ms
        for w in range(W):
            mw = rmax(ms_cur)
            ew = ms_cur == mw
            m.append(mw)
            e.append(ew)
            if w < W - 1:
                ms_cur = jnp.where(ew, -jnp.inf, ms_cur)
                c.append(rsum(ew.astype(jnp.int32)))
            j.append(rmin(jnp.where(ew, lin, BIGI)))
            b.append(picks(j[w]))

        a = [m[0] > -jnp.inf]
        for w in range(1, W):
            acc = a[w - 1] & (c[w - 1] == one) & (m[w] > -jnp.inf)
            for v in range(w):
                acc = acc & jnp.logical_not(piou(b[v], b[w]) > THR)
            a.append(acc)

        def srow(a_w, j_w, b_w):
            xx1 = jnp.maximum(b_w[0], x1)
            yy1 = jnp.maximum(b_w[1], y1)
            xx2 = jnp.minimum(b_w[2], x2)
            yy2 = jnp.minimum(b_w[3], y2)
            inter = jnp.maximum(xx2 - xx1, 0.0) * jnp.maximum(yy2 - yy1, 0.0)
            iou = inter / (b_w[4] + area_ref[...] - inter + 1e-9)
            return a_w & ((iou > THR) | (lin == j_w))

        sup = srow(a[0], j[0], b[0])
        for w in range(1, W):
            sup = sup | srow(a[w], j[w], b[w])
        ms_ref[...] = jnp.where(sup, -jnp.inf, ms)

        for w in range(W):
            p = pl.ds(cnt + w, 1)
            ox1[p, :] = jnp.where(a[w], b[w][0], 0.0)
            oy1[p, :] = jnp.where(a[w], b[w][1], 0.0)
            ox2[p, :] = jnp.where(a[w], b[w][2], 0.0)
            oy2[p, :] = jnp.where(a[w], b[w][3], 0.0)
            osc[p, :] = jnp.where(a[w], m[w], 0.0)

        ka = a[0].astype(jnp.int32)
        for w in range(1, W):
            ka = ka + a[w].astype(jnp.int32)
        ka = ka[0, 0]
        return cnt + ka, ka == 0

    def cond(carry):
        cnt, done = carry
        return jnp.logical_and(cnt < K_POST, jnp.logical_not(done))

    lax.while_loop(cond, step, (jnp.int32(0), False))


def kernel(boxes, scores):
    s2 = jnp.pad(scores, (0, PAD - N), constant_values=-jnp.inf).reshape(R, 128)
    bx = jnp.pad(boxes, ((0, PAD - N), (0, 0)))
    aos = jnp.concatenate(
        [bx, s2.reshape(PAD, 1), jnp.zeros((PAD, 123), jnp.float32)], axis=1)

    dest = pl.pallas_call(
        _select_body,
        out_shape=jax.ShapeDtypeStruct((R, 128), jnp.int32),
    )(s2)

    mesh = plsc.VectorSubcoreMesh(core_axis_name="c", subcore_axis_name="s")
    compact = functools.partial(
        pl.kernel,
        out_type=jax.ShapeDtypeStruct((OUT_ROWS, 128), jnp.float32),
        scratch_types=[
            pltpu.VMEM((CB, 128, 128), jnp.float32),
            pltpu.VMEM((CB, 128), jnp.int32),
            pltpu.SemaphoreType.DMA,
        ],
        mesh=mesh,
    )(_compact_body)
    crows = compact(
        aos.reshape(NW, CB, 128, 128),
        dest.reshape(NW, CB, 128),
    )[:C_PAD]

    planes = [crows[:, i].reshape(RC, 128) for i in range(5)]
    careas = jnp.maximum(planes[2] - planes[0], 0.0) * jnp.maximum(
        planes[3] - planes[1], 0.0)
    packed = jnp.stack(planes[:4] + [careas], axis=1).reshape(5 * RC, 128)

    outs = pl.pallas_call(
        _nms_body,
        out_shape=[jax.ShapeDtypeStruct((K_POST + W, 1), jnp.float32)] * 5,
        scratch_shapes=[
            pltpu.VMEM((RC, 128), jnp.float32),
            pltpu.VMEM((RC, 128), jnp.float32),
            pltpu.VMEM((RC, 128), jnp.int32),
        ],
    )(*planes, packed)
    return jnp.concatenate([o[:K_POST] for o in outs], axis=1)
